# baseline (speedup 1.0000x reference)
import jax
import jax.numpy as jnp
from jax.experimental import pallas as pl
from jax.experimental.pallas import tpu as pltpu


def _csq_kernel(c_ref, csq_ref):
    c = c_ref[...]
    csq_ref[...] = jnp.sum(c * c, axis=1, keepdims=True)


def _vq_kernel(x_ref, wt_ref, b_ref, c_ref, csq_ref, out_ref):
    x = x_ref[0]
    c = c_ref[...]
    zT = jnp.dot(wt_ref[...], x, preferred_element_type=jnp.float32)
    zT = zT + b_ref[...]
    y2 = jnp.dot(c, zT + zT, preferred_element_type=jnp.float32)
    z_sq = jnp.sum(zT * zT, axis=0, keepdims=True)
    dist = (z_sq - y2) + csq_ref[...]
    codes = jnp.argmin(dist, axis=0)
    out_ref[...] = codes[None, None, :].astype(jnp.int32)


def kernel(ssl_content, W, b, codebook):
    B, IN_DIM, T = ssl_content.shape
    K, CODE_DIM = codebook.shape
    Tt = 1024
    nT = T // Tt
    Wt = W.T
    b2 = b.reshape(CODE_DIM, 1)

    csq = pl.pallas_call(
        _csq_kernel,
        out_shape=jax.ShapeDtypeStruct((K, 1), jnp.float32),
    )(codebook)

    out = pl.pallas_call(
        _vq_kernel,
        grid=(B, nT),
        in_specs=[
            pl.BlockSpec((1, IN_DIM, Tt), lambda i, j: (i, 0, j)),
            pl.BlockSpec((CODE_DIM, IN_DIM), lambda i, j: (0, 0)),
            pl.BlockSpec((CODE_DIM, 1), lambda i, j: (0, 0)),
            pl.BlockSpec((K, CODE_DIM), lambda i, j: (0, 0)),
            pl.BlockSpec((K, 1), lambda i, j: (0, 0)),
        ],
        out_specs=pl.BlockSpec((1, 1, Tt), lambda i, j: (i, 0, j)),
        out_shape=jax.ShapeDtypeStruct((B, 1, T), jnp.int32),
    )(ssl_content, Wt, b2, codebook, csq)
    return out.reshape(B, T)


# speedup vs baseline: 1.0839x; 1.0839x over previous
import jax
import jax.numpy as jnp
from jax.experimental import pallas as pl
from jax.experimental.pallas import tpu as pltpu


def _vq_kernel(x_ref, wt_ref, b_ref, c_ref, out_ref):
    x = x_ref[0]
    c = c_ref[...]
    zT = jnp.dot(wt_ref[...], x, preferred_element_type=jnp.float32)
    zT = zT + b_ref[...]
    y2 = jnp.dot(c, zT + zT, preferred_element_type=jnp.float32)
    z_sq = jnp.sum(zT * zT, axis=0, keepdims=True)
    c_sq = jnp.sum(c * c, axis=1, keepdims=True)
    dist = (z_sq - y2) + c_sq

    K_, Tt = dist.shape
    G = 8
    bv = dist[0:G]
    bi = jnp.zeros((G, Tt), jnp.int32)
    for r in range(1, K_ // G):
        cur = dist[r * G:(r + 1) * G]
        m = cur < bv
        bv = jnp.where(m, cur, bv)
        bi = jnp.where(m, jnp.int32(r), bi)
    mcol = jnp.min(bv, axis=0, keepdims=True)
    s_iota = jax.lax.broadcasted_iota(jnp.int32, (G, Tt), 0)
    kcand = jnp.where(bv == mcol, bi * G + s_iota, jnp.int32(K_))
    codes = jnp.min(kcand, axis=0)
    out_ref[...] = codes[None, None, :]


def kernel(ssl_content, W, b, codebook):
    B, IN_DIM, T = ssl_content.shape
    K, CODE_DIM = codebook.shape
    Tt = 1024
    nT = T // Tt
    Wt = W.T
    b2 = b.reshape(CODE_DIM, 1)

    out = pl.pallas_call(
        _vq_kernel,
        grid=(B, nT),
        in_specs=[
            pl.BlockSpec((1, IN_DIM, Tt), lambda i, j: (i, 0, j)),
            pl.BlockSpec((CODE_DIM, IN_DIM), lambda i, j: (0, 0)),
            pl.BlockSpec((CODE_DIM, 1), lambda i, j: (0, 0)),
            pl.BlockSpec((K, CODE_DIM), lambda i, j: (0, 0)),
        ],
        out_specs=pl.BlockSpec((1, 1, Tt), lambda i, j: (i, 0, j)),
        out_shape=jax.ShapeDtypeStruct((B, 1, T), jnp.int32),
    )(ssl_content, Wt, b2, codebook)
    return out.reshape(B, T)
